# baseline (speedup 1.0000x reference)
import functools

import jax
import jax.numpy as jnp
from jax import lax
from jax.experimental import pallas as pl
from jax.experimental.pallas import tpu as pltpu
from jax.experimental.pallas import tpu_sc as plsc

B, N, P, K, D, H = 4, 8192, 8192, 16, 128, 64
L = 16
NC, NS = 2, 16
NW = NC * NS
WPB = NW // B
BN = B * N
BP = B * P
PW = BP // NW
W = 8
CW = W * K
NCHUNK = PW // W
NBUF = 4
STG = 128
BLK = 8192
NBLK = N // BLK


def _score_body(x_ref, w1_ref, b1_ref, v_ref, bv_ref, y_ref, sv_ref):
    xb = x_ref[0]
    h = jnp.tanh(
        jnp.dot(xb.astype(jnp.bfloat16), w1_ref[...].astype(jnp.bfloat16),
                preferred_element_type=jnp.float32) + b1_ref[...])
    logit = jnp.dot(h, v_ref[...],
                    preferred_element_type=jnp.float32) + bv_ref[...]
    s = jax.nn.sigmoid(logit)
    ht = jnp.swapaxes(h, 0, 1)
    logit_row = jnp.dot(jnp.swapaxes(v_ref[...], 0, 1), ht,
                        preferred_element_type=jnp.float32) + bv_ref[...]
    sv_ref[...] = jax.nn.sigmoid(logit_row)[0]
    y_ref[...] = xb * s


def _scores_and_table(x, W1, b1, V, bV):
    y, sv = pl.pallas_call(
        _score_body,
        grid=(B, NBLK),
        in_specs=[
            pl.BlockSpec((1, BLK, D), lambda b, i: (b, i, 0)),
            pl.BlockSpec((D, H), lambda b, i: (0, 0)),
            pl.BlockSpec((1, H), lambda b, i: (0, 0)),
            pl.BlockSpec((H, 1), lambda b, i: (0, 0)),
            pl.BlockSpec((1, 1), lambda b, i: (0, 0)),
        ],
        out_specs=[
            pl.BlockSpec((BLK, D), lambda b, i: (b * NBLK + i, 0)),
            pl.BlockSpec((BLK,), lambda b, i: (b * NBLK + i,)),
        ],
        out_shape=[
            jax.ShapeDtypeStruct((BN, D), jnp.float32),
            jax.ShapeDtypeStruct((BN,), jnp.float32),
        ],
    )(x, W1, b1.reshape(1, H), V, bV.reshape(1, 1))
    return y, sv


_VECTOR_MESH = plsc.VectorSubcoreMesh(core_axis_name="c", subcore_axis_name="s")


@functools.partial(
    pl.kernel,
    out_type=jax.ShapeDtypeStruct((BP, D), jnp.float32),
    mesh=_VECTOR_MESH,
    compiler_params=pltpu.CompilerParams(needs_layout_passes=False),
    scratch_types=(
        [pltpu.VMEM((K, STG), jnp.int32),
         pltpu.VMEM((PW * K,), jnp.int32),
         pltpu.VMEM((N,), jnp.float32)]
        + [pltpu.VMEM((CW, D), jnp.float32)] * NBUF
        + [pltpu.VMEM((NBUF * W, D), jnp.float32)] * 2
        + [pltpu.SemaphoreType.DMA] * (NBUF + 2)
    ),
)
def _pool(y_hbm, idx_hbm, xs_hbm, out_hbm, idx_v, idx_f, sv, *bufs):
    rows = bufs[:NBUF]
    outs = bufs[NBUF:NBUF + 2]
    gss = bufs[NBUF + 2:2 * NBUF + 2]
    oss = bufs[2 * NBUF + 2:2 * NBUF + 4]
    cid = lax.axis_index("c")
    sid = lax.axis_index("s")
    wid = sid * NC + cid
    b = wid // WPB
    p0 = (wid % WPB) * PW
    pbase = wid * PW
    rowbase = b * N

    iota = lax.iota(jnp.int32, L)
    iK = iota * K

    def stage(g):
        pltpu.sync_copy(idx_hbm.at[b, :, pl.ds(p0 + g * STG, STG)], idx_v)

        @pl.loop(0, K)
        def _(k):
            for t in range(STG // L):
                vals = idx_v[k, pl.ds(t * L, L)] + rowbase
                offs = iK + ((g * STG + t * L) * K + k)
                plsc.store_scatter(idx_f, [offs], vals)

    def fire_gather(c, rbuf, sem):
        pltpu.async_copy(y_hbm.at[idx_f.at[pl.ds(c * CW, CW)]], rbuf, sem)

    def wait_gather(rbuf, sem):
        pltpu.make_async_copy(y_hbm.at[idx_f.at[pl.ds(0, CW)]], rbuf,
                              sem).wait()

    def compute(c, rbuf, obuf, ooff):
        @pl.loop(0, W)
        def _(w):
            r0 = w * K
            iloc = idx_f[pl.ds((c * W + w) * K, K)] - rowbase
            svec = plsc.load_gather(sv, [iloc])
            ssum = jnp.sum(svec)
            accs = [rbuf[r0, pl.ds(j * L, L)] for j in range(D // L)]
            for k in range(1, K):
                accs = [a + rbuf[r0 + k, pl.ds(j * L, L)]
                        for j, a in enumerate(accs)]
            for j in range(D // L):
                obuf[ooff + w, pl.ds(j * L, L)] = accs[j] / ssum

    def wait_out(obuf, sem):
        pltpu.make_async_copy(obuf, out_hbm.at[pl.ds(pbase, NBUF * W)],
                              sem).wait()

    stage(0)
    for j in range(NBUF):
        fire_gather(j, rows[j], gss[j])
    pltpu.sync_copy(xs_hbm.at[pl.ds(rowbase, N)], sv)

    @pl.loop(1, PW // STG, step=1)
    def _(g):
        stage(g)

    @pl.loop(0, NCHUNK, step=2 * NBUF)
    def _(c):
        for half in range(2):
            cbase = c + half * NBUF

            @pl.when(c >= 2 * NBUF)
            def _():
                wait_out(outs[half], oss[half])

            for j in range(NBUF):
                cc = cbase + j
                wait_gather(rows[j], gss[j])
                compute(cc, rows[j], outs[half], j * W)

                @pl.when(cc + NBUF < NCHUNK)
                def _():
                    fire_gather(cc + NBUF, rows[j], gss[j])

            pltpu.async_copy(
                outs[half],
                out_hbm.at[pl.ds(pbase + cbase * W, NBUF * W)], oss[half])

    for half in range(2):
        wait_out(outs[half], oss[half])


def kernel(x, idx, W1, b1, V, bV):
    y, sv = _scores_and_table(x, W1, b1, V, bV)
    idx_t = jnp.swapaxes(idx.astype(jnp.int32), 1, 2)
    out = _pool(y, idx_t, sv)
    return out.reshape(B, P, D), sv.reshape(B, N, 1)


# speedup vs baseline: 1.0066x; 1.0066x over previous
import functools

import jax
import jax.numpy as jnp
from jax import lax
from jax.experimental import pallas as pl
from jax.experimental.pallas import tpu as pltpu
from jax.experimental.pallas import tpu_sc as plsc

B, N, P, K, D, H = 4, 8192, 8192, 16, 128, 64
L = 16
NC, NS = 2, 16
NW = NC * NS
WPB = NW // B
BN = B * N
BP = B * P
PW = BP // NW
W = 8
CW = W * K
NCHUNK = PW // W
NBUF = 4
STG = 128
BLK = 8192
NBLK = N // BLK


def _score_body(x_ref, w1_ref, b1_ref, v_ref, bv_ref, y_ref, sv_ref):
    xb = x_ref[0]
    h = jnp.tanh(
        jnp.dot(xb.astype(jnp.bfloat16), w1_ref[...].astype(jnp.bfloat16),
                preferred_element_type=jnp.float32) + b1_ref[...])
    logit = jnp.dot(h, v_ref[...],
                    preferred_element_type=jnp.float32) + bv_ref[...]
    s = jax.nn.sigmoid(logit)
    ht = jnp.swapaxes(h, 0, 1)
    logit_row = jnp.dot(jnp.swapaxes(v_ref[...], 0, 1), ht,
                        preferred_element_type=jnp.float32) + bv_ref[...]
    sv_ref[...] = jax.nn.sigmoid(logit_row)[0]
    y_ref[...] = xb * s


def _scores_and_table(x, W1, b1, V, bV):
    y, sv = pl.pallas_call(
        _score_body,
        grid=(B, NBLK),
        in_specs=[
            pl.BlockSpec((1, BLK, D), lambda b, i: (b, i, 0)),
            pl.BlockSpec((D, H), lambda b, i: (0, 0)),
            pl.BlockSpec((1, H), lambda b, i: (0, 0)),
            pl.BlockSpec((H, 1), lambda b, i: (0, 0)),
            pl.BlockSpec((1, 1), lambda b, i: (0, 0)),
        ],
        out_specs=[
            pl.BlockSpec((BLK, D), lambda b, i: (b * NBLK + i, 0)),
            pl.BlockSpec((BLK,), lambda b, i: (b * NBLK + i,)),
        ],
        out_shape=[
            jax.ShapeDtypeStruct((BN, D), jnp.float32),
            jax.ShapeDtypeStruct((BN,), jnp.float32),
        ],
    )(x, W1, b1.reshape(1, H), V, bV.reshape(1, 1))
    return y, sv


_VECTOR_MESH = plsc.VectorSubcoreMesh(core_axis_name="c", subcore_axis_name="s")


@functools.partial(
    pl.kernel,
    out_type=jax.ShapeDtypeStruct((BP, D), jnp.float32),
    mesh=_VECTOR_MESH,
    compiler_params=pltpu.CompilerParams(needs_layout_passes=False),
    scratch_types=(
        [pltpu.VMEM((K, STG), jnp.int32),
         pltpu.VMEM((PW * K,), jnp.int32),
         pltpu.VMEM((N,), jnp.float32)]
        + [pltpu.VMEM((CW, D), jnp.float32)] * NBUF
        + [pltpu.VMEM((W, D), jnp.float32)] * NBUF
        + [pltpu.SemaphoreType.DMA] * (2 * NBUF)
    ),
)
def _pool(y_hbm, idx_hbm, xs_hbm, out_hbm, idx_v, idx_f, sv, *bufs):
    rows = bufs[:NBUF]
    outs = bufs[NBUF:2 * NBUF]
    gss = bufs[2 * NBUF:3 * NBUF]
    oss = bufs[3 * NBUF:4 * NBUF]
    cid = lax.axis_index("c")
    sid = lax.axis_index("s")
    wid = sid * NC + cid
    b = wid // WPB
    p0 = (wid % WPB) * PW
    pbase = wid * PW
    rowbase = b * N

    iota = lax.iota(jnp.int32, L)
    iK = iota * K

    def stage(g):
        pltpu.sync_copy(idx_hbm.at[b, :, pl.ds(p0 + g * STG, STG)], idx_v)

        @pl.loop(0, K)
        def _(k):
            for t in range(STG // L):
                vals = idx_v[k, pl.ds(t * L, L)] + rowbase
                offs = iK + ((g * STG + t * L) * K + k)
                plsc.store_scatter(idx_f, [offs], vals)

    def fire_gather(c, rbuf, sem):
        pltpu.async_copy(y_hbm.at[idx_f.at[pl.ds(c * CW, CW)]], rbuf, sem)

    def wait_gather(rbuf, sem):
        pltpu.make_async_copy(y_hbm.at[idx_f.at[pl.ds(0, CW)]], rbuf,
                              sem).wait()

    def compute(c, rbuf, obuf):
        @pl.loop(0, W)
        def _(w):
            r0 = w * K
            iloc = idx_f[pl.ds((c * W + w) * K, K)] - rowbase
            svec = plsc.load_gather(sv, [iloc])
            ssum = jnp.sum(svec)
            accs = [rbuf[r0, pl.ds(j * L, L)] for j in range(D // L)]
            for k in range(1, K):
                accs = [a + rbuf[r0 + k, pl.ds(j * L, L)]
                        for j, a in enumerate(accs)]
            for j in range(D // L):
                obuf[w, pl.ds(j * L, L)] = accs[j] / ssum

    def wait_out(obuf, sem):
        pltpu.make_async_copy(obuf, out_hbm.at[pl.ds(pbase, W)], sem).wait()

    stage(0)
    for j in range(NBUF):
        fire_gather(j, rows[j], gss[j])
    pltpu.sync_copy(xs_hbm.at[pl.ds(rowbase, N)], sv)

    @pl.loop(1, PW // STG, step=1)
    def _(g):
        stage(g)

    @pl.loop(0, NCHUNK, step=NBUF)
    def _(c):
        for j in range(NBUF):
            cc = c + j
            wait_gather(rows[j], gss[j])

            @pl.when(c >= NBUF)
            def _():
                wait_out(outs[j], oss[j])

            compute(cc, rows[j], outs[j])

            @pl.when(cc + NBUF < NCHUNK)
            def _():
                fire_gather(cc + NBUF, rows[j], gss[j])

            pltpu.async_copy(outs[j], out_hbm.at[pl.ds(pbase + cc * W, W)],
                             oss[j])

    for j in range(NBUF):
        wait_out(outs[j], oss[j])


def kernel(x, idx, W1, b1, V, bV):
    y, sv = _scores_and_table(x, W1, b1, V, bV)
    idx_t = jnp.swapaxes(idx.astype(jnp.int32), 1, 2)
    out = _pool(y, idx_t, sv)
    return out.reshape(B, P, D), sv.reshape(B, N, 1)
